# Initial kernel scaffold; baseline (speedup 1.0000x reference)
import jax
import jax.numpy as jnp
from jax.experimental import pallas as pl


def kernel(x, edge_index, W1, b1, W2, b2):
    raise NotImplementedError("write your pallas kernel here")


# speedup vs baseline: 31.0168x; 31.0168x over previous
import functools

import jax
import jax.numpy as jnp
from jax import lax
from jax.experimental import pallas as pl
from jax.experimental.pallas import tpu as pltpu
from jax.experimental.pallas import tpu_sc as plsc

N = 10000
E = 320000
D = 128
F = 16

NC = 2
NS = 16
NW = NC * NS

CH = 128
TCH = 80
E_PAD = NW * TCH * CH
NCH_TOT = E_PAD // CH
N_PAD = 10112
STRIPE = N_PAD // NS

_mesh = plsc.VectorSubcoreMesh(core_axis_name="c", subcore_axis_name="s")
_sc_params = pltpu.CompilerParams(use_tc_tiling_on_sc=False)


@functools.partial(
    pl.kernel,
    out_type=jax.ShapeDtypeStruct((NC * N_PAD,), jnp.float32),
    mesh=_mesh,
    scratch_types=[
        pltpu.VMEM((TCH, CH), jnp.int32),
        pltpu.VMEM((CH,), jnp.float32),
        pltpu.VMEM((STRIPE + 8,), jnp.float32),
        pltpu.VMEM_SHARED((N_PAD,), jnp.float32),
    ],
    compiler_params=_sc_params,
)
def _deg_kernel(dst_hbm, out_hbm, idx_v, ones_v, stage_v, acc_sh):
    c = lax.axis_index("c")
    s = lax.axis_index("s")
    w = c * NS + s
    for k in range((STRIPE + 8) // 16):
        stage_v[pl.ds(k * 16, 16)] = jnp.zeros((16,), jnp.float32)
    for k in range(CH // 16):
        ones_v[pl.ds(k * 16, 16)] = jnp.full((16,), 1.0, jnp.float32)
    pltpu.sync_copy(stage_v.at[pl.ds(0, STRIPE)],
                    acc_sh.at[pl.ds(s * STRIPE, STRIPE)])
    pltpu.sync_copy(dst_hbm.at[pl.ds(w * TCH, TCH)], idx_v)
    plsc.subcore_barrier()

    def body(j, carry):
        pltpu.sync_copy(ones_v, acc_sh.at[idx_v.at[j]], add=True)
        return carry

    lax.fori_loop(0, TCH, body, 0)
    plsc.subcore_barrier()
    pltpu.sync_copy(acc_sh.at[pl.ds(s * STRIPE, STRIPE)],
                    stage_v.at[pl.ds(0, STRIPE)])
    pltpu.sync_copy(stage_v.at[pl.ds(0, STRIPE)],
                    out_hbm.at[pl.ds(c * N_PAD + s * STRIPE, STRIPE)])


@functools.partial(
    pl.kernel,
    out_type=jax.ShapeDtypeStruct((NC, N_PAD, F), jnp.float32),
    mesh=_mesh,
    scratch_types=[
        pltpu.VMEM((TCH, CH), jnp.int32),
        pltpu.VMEM((TCH, CH), jnp.int32),
        pltpu.VMEM((CH, F), jnp.float32),
        pltpu.VMEM((STRIPE, F), jnp.float32),
        pltpu.VMEM_SHARED((N_PAD, F), jnp.float32),
        pltpu.SemaphoreType.DMA,
    ],
    compiler_params=_sc_params,
)
def _scat_kernel(g_hbm, src_hbm, dst_hbm, out_hbm,
                 sidx, didx, rows, stage_v, acc_sh, gsem):
    c = lax.axis_index("c")
    s = lax.axis_index("s")
    w = c * NS + s

    def zbody(i, carry):
        stage_v[i, :] = jnp.zeros((F,), jnp.float32)
        return carry

    lax.fori_loop(0, STRIPE, zbody, 0)
    pltpu.sync_copy(stage_v, acc_sh.at[pl.ds(s * STRIPE, STRIPE)])
    pltpu.sync_copy(src_hbm.at[pl.ds(w * TCH, TCH)], sidx)
    pltpu.sync_copy(dst_hbm.at[pl.ds(w * TCH, TCH)], didx)
    plsc.subcore_barrier()

    def body(j, carry):
        pltpu.async_copy(g_hbm.at[sidx.at[j]], rows, gsem).wait()
        pltpu.sync_copy(rows, acc_sh.at[didx.at[j]], add=True)
        return carry

    lax.fori_loop(0, TCH, body, 0)
    plsc.subcore_barrier()
    pltpu.sync_copy(acc_sh.at[pl.ds(s * STRIPE, STRIPE)], stage_v)
    pltpu.sync_copy(stage_v, out_hbm.at[c, pl.ds(s * STRIPE, STRIPE), :])


def _prep_body(deg_ref, x_ref, w1_ref, g_ref, dinv_ref):
    deg = deg_ref[0, :] + deg_ref[1, :] + 1.0
    dinv = lax.rsqrt(deg)
    dinv_ref[...] = dinv
    h = jnp.dot(x_ref[...], w1_ref[...], preferred_element_type=jnp.float32)
    g_ref[...] = h * dinv[:, None]


def _mid_body(s_ref, g_ref, dinv_ref, b_ref, w2_ref, out_ref):
    tot = (s_ref[0] + s_ref[1] + g_ref[...]) * dinv_ref[...][:, None]
    h = jnp.maximum(tot + b_ref[...][None, :], 0.0)
    out_ref[...] = jnp.dot(h, w2_ref[...],
                           preferred_element_type=jnp.float32) * dinv_ref[...][:, None]


def _final_body(s_ref, g_ref, dinv_ref, b_ref, out_ref):
    logits = (s_ref[0] + s_ref[1] + g_ref[...]) * dinv_ref[...][:, None]
    logits = logits + b_ref[...][None, :]
    m = jnp.max(logits, axis=1, keepdims=True)
    z = logits - m
    out_ref[...] = z - jnp.log(jnp.sum(jnp.exp(z), axis=1, keepdims=True))


_prep_call = pl.pallas_call(
    _prep_body,
    out_shape=(jax.ShapeDtypeStruct((N_PAD, F), jnp.float32),
               jax.ShapeDtypeStruct((N_PAD,), jnp.float32)),
)

_mid_call = pl.pallas_call(
    _mid_body,
    out_shape=jax.ShapeDtypeStruct((N_PAD, F), jnp.float32),
)

_final_call = pl.pallas_call(
    _final_body,
    out_shape=jax.ShapeDtypeStruct((N_PAD, F), jnp.float32),
)


def kernel(x, edge_index, W1, b1, W2, b2):
    src = edge_index[0]
    dst = edge_index[1]
    pad = jnp.full((E_PAD - E,), N, jnp.int32)
    src_p = jnp.concatenate([src, pad]).reshape(NCH_TOT, CH)
    dst_p = jnp.concatenate([dst, pad]).reshape(NCH_TOT, CH)
    x_p = jnp.pad(x, ((0, N_PAD - N), (0, 0)))

    deg2 = _deg_kernel(dst_p).reshape(NC, N_PAD)
    g1, dinv = _prep_call(deg2, x_p, W1)
    s1 = _scat_kernel(g1, src_p, dst_p)
    g2 = _mid_call(s1, g1, dinv, b1, W2)
    s2 = _scat_kernel(g2, src_p, dst_p)
    out = _final_call(s2, g2, dinv, b2)
    return out[:N]


# speedup vs baseline: 42.0021x; 1.3542x over previous
import functools

import jax
import jax.numpy as jnp
from jax import lax
from jax.experimental import pallas as pl
from jax.experimental.pallas import tpu as pltpu
from jax.experimental.pallas import tpu_sc as plsc

N = 10000
E = 320000
D = 128
F = 16

NC = 2
NS = 16
NW = NC * NS

CH = 128
TCH = 80
E_PAD = NW * TCH * CH
NCH_TOT = E_PAD // CH
N_PAD = 10112
STRIPE = N_PAD // NS

_mesh = plsc.VectorSubcoreMesh(core_axis_name="c", subcore_axis_name="s")
_sc_params = pltpu.CompilerParams(use_tc_tiling_on_sc=False)


@functools.partial(
    pl.kernel,
    out_type=jax.ShapeDtypeStruct((NC * N_PAD,), jnp.float32),
    mesh=_mesh,
    scratch_types=[
        pltpu.VMEM((TCH, CH), jnp.int32),
        pltpu.VMEM((CH,), jnp.float32),
        pltpu.VMEM((STRIPE + 8,), jnp.float32),
        pltpu.VMEM_SHARED((N_PAD,), jnp.float32),
    ],
    compiler_params=_sc_params,
)
def _deg_kernel(dst_hbm, out_hbm, idx_v, ones_v, stage_v, acc_sh):
    c = lax.axis_index("c")
    s = lax.axis_index("s")
    w = c * NS + s
    for k in range((STRIPE + 8) // 16):
        stage_v[pl.ds(k * 16, 16)] = jnp.zeros((16,), jnp.float32)
    for k in range(CH // 16):
        ones_v[pl.ds(k * 16, 16)] = jnp.full((16,), 1.0, jnp.float32)
    pltpu.sync_copy(stage_v.at[pl.ds(0, STRIPE)],
                    acc_sh.at[pl.ds(s * STRIPE, STRIPE)])
    pltpu.sync_copy(dst_hbm.at[pl.ds(w * TCH, TCH)], idx_v)
    plsc.subcore_barrier()

    def body(j, carry):
        pltpu.sync_copy(ones_v, acc_sh.at[idx_v.at[j]], add=True)
        return carry

    lax.fori_loop(0, TCH, body, 0)
    plsc.subcore_barrier()
    pltpu.sync_copy(acc_sh.at[pl.ds(s * STRIPE, STRIPE)],
                    stage_v.at[pl.ds(0, STRIPE)])
    pltpu.sync_copy(stage_v.at[pl.ds(0, STRIPE)],
                    out_hbm.at[pl.ds(c * N_PAD + s * STRIPE, STRIPE)])


NBUF = 4
OUTER = TCH // NBUF


@functools.partial(
    pl.kernel,
    out_type=jax.ShapeDtypeStruct((NC, N_PAD, F), jnp.float32),
    mesh=_mesh,
    scratch_types=[
        pltpu.VMEM((TCH, CH), jnp.int32),
        pltpu.VMEM((TCH, CH), jnp.int32),
        pltpu.VMEM((NBUF, CH, F), jnp.float32),
        pltpu.VMEM((STRIPE, F), jnp.float32),
        pltpu.VMEM_SHARED((N_PAD, F), jnp.float32),
        pltpu.SemaphoreType.DMA,
    ] + [pltpu.SemaphoreType.DMA] * (2 * NBUF),
    compiler_params=_sc_params,
)
def _scat_kernel(g_hbm, src_hbm, dst_hbm, out_hbm,
                 sidx, didx, rows, stage_v, acc_sh, isem, *bsems):
    gsems = bsems[:NBUF]
    ssems = bsems[NBUF:]
    c = lax.axis_index("c")
    s = lax.axis_index("s")
    w = c * NS + s

    icopy_s = pltpu.async_copy(src_hbm.at[pl.ds(w * TCH, TCH)], sidx, isem)
    icopy_d = pltpu.async_copy(dst_hbm.at[pl.ds(w * TCH, TCH)], didx, isem)
    zv = jnp.zeros((F,), jnp.float32)

    def zbody(i, carry):
        for r in range(8):
            stage_v[i * 8 + r, :] = zv
        return carry

    lax.fori_loop(0, STRIPE // 8, zbody, 0)
    pltpu.sync_copy(stage_v, acc_sh.at[pl.ds(s * STRIPE, STRIPE)])
    icopy_s.wait()
    icopy_d.wait()
    plsc.subcore_barrier()

    for b in range(NBUF):
        pltpu.async_copy(g_hbm.at[sidx.at[b]], rows.at[b], gsems[b])

    def body(jj, carry):
        for b in range(NBUF):
            j = jj * NBUF + b
            pltpu.make_async_copy(g_hbm.at[sidx.at[j]], rows.at[b],
                                  gsems[b]).wait()
            pltpu.async_copy(rows.at[b], acc_sh.at[didx.at[j]], ssems[b],
                             add=True)
            pltpu.make_async_copy(rows.at[b], acc_sh.at[didx.at[j]],
                                  ssems[b]).wait()
            pltpu.async_copy(g_hbm.at[sidx.at[j + NBUF]], rows.at[b],
                             gsems[b])
        return carry

    lax.fori_loop(0, OUTER - 1, body, 0)
    for b in range(NBUF):
        j = (OUTER - 1) * NBUF + b
        pltpu.make_async_copy(g_hbm.at[sidx.at[j]], rows.at[b],
                              gsems[b]).wait()
        pltpu.async_copy(rows.at[b], acc_sh.at[didx.at[j]], ssems[b],
                         add=True)
    for b in range(NBUF):
        j = (OUTER - 1) * NBUF + b
        pltpu.make_async_copy(rows.at[b], acc_sh.at[didx.at[j]],
                              ssems[b]).wait()
    plsc.subcore_barrier()
    pltpu.sync_copy(acc_sh.at[pl.ds(s * STRIPE, STRIPE)], stage_v)
    pltpu.sync_copy(stage_v, out_hbm.at[c, pl.ds(s * STRIPE, STRIPE), :])


def _prep_body(deg_ref, x_ref, w1_ref, g_ref, dinv_ref):
    deg = deg_ref[0, :] + deg_ref[1, :] + 1.0
    dinv = lax.rsqrt(deg)
    dinv_ref[...] = dinv
    h = jnp.dot(x_ref[...], w1_ref[...], preferred_element_type=jnp.float32)
    g_ref[...] = h * dinv[:, None]


def _mid_body(s_ref, g_ref, dinv_ref, b_ref, w2_ref, out_ref):
    tot = (s_ref[0] + s_ref[1] + g_ref[...]) * dinv_ref[...][:, None]
    h = jnp.maximum(tot + b_ref[...][None, :], 0.0)
    out_ref[...] = jnp.dot(h, w2_ref[...],
                           preferred_element_type=jnp.float32) * dinv_ref[...][:, None]


def _final_body(s_ref, g_ref, dinv_ref, b_ref, out_ref):
    logits = (s_ref[0] + s_ref[1] + g_ref[...]) * dinv_ref[...][:, None]
    logits = logits + b_ref[...][None, :]
    m = jnp.max(logits, axis=1, keepdims=True)
    z = logits - m
    out_ref[...] = z - jnp.log(jnp.sum(jnp.exp(z), axis=1, keepdims=True))


_prep_call = pl.pallas_call(
    _prep_body,
    out_shape=(jax.ShapeDtypeStruct((N_PAD, F), jnp.float32),
               jax.ShapeDtypeStruct((N_PAD,), jnp.float32)),
)

_mid_call = pl.pallas_call(
    _mid_body,
    out_shape=jax.ShapeDtypeStruct((N_PAD, F), jnp.float32),
)

_final_call = pl.pallas_call(
    _final_body,
    out_shape=jax.ShapeDtypeStruct((N_PAD, F), jnp.float32),
)


def kernel(x, edge_index, W1, b1, W2, b2):
    src = edge_index[0]
    dst = edge_index[1]
    pad = jnp.full((E_PAD - E,), N, jnp.int32)
    src_p = jnp.concatenate([src, pad]).reshape(NCH_TOT, CH)
    dst_p = jnp.concatenate([dst, pad]).reshape(NCH_TOT, CH)
    x_p = jnp.pad(x, ((0, N_PAD - N), (0, 0)))

    deg2 = _deg_kernel(dst_p).reshape(NC, N_PAD)
    g1, dinv = _prep_call(deg2, x_p, W1)
    s1 = _scat_kernel(g1, src_p, dst_p)
    g2 = _mid_call(s1, g1, dinv, b1, W2)
    s2 = _scat_kernel(g2, src_p, dst_p)
    out = _final_call(s2, g2, dinv, b2)
    return out[:N]


# speedup vs baseline: 63.4490x; 1.5106x over previous
import functools

import jax
import jax.numpy as jnp
from jax import lax
from jax.experimental import pallas as pl
from jax.experimental.pallas import tpu as pltpu
from jax.experimental.pallas import tpu_sc as plsc

N = 10000
E = 320000
D = 128
F = 16

NC = 2
NS = 16
NW = NC * NS

CH = 128
TCH = 80
E_PAD = NW * TCH * CH
NCH_TOT = E_PAD // CH
N_PAD = 10112
STRIPE = N_PAD // NS

_mesh = plsc.VectorSubcoreMesh(core_axis_name="c", subcore_axis_name="s")
_sc_params = pltpu.CompilerParams(use_tc_tiling_on_sc=False)


@functools.partial(
    pl.kernel,
    out_type=jax.ShapeDtypeStruct((NC * N_PAD,), jnp.float32),
    mesh=_mesh,
    scratch_types=[
        pltpu.VMEM((TCH, CH), jnp.int32),
        pltpu.VMEM((CH,), jnp.float32),
        pltpu.VMEM((STRIPE + 8,), jnp.float32),
        pltpu.VMEM_SHARED((N_PAD,), jnp.float32),
    ],
    compiler_params=_sc_params,
)
def _deg_kernel(dst_hbm, out_hbm, idx_v, ones_v, stage_v, acc_sh):
    c = lax.axis_index("c")
    s = lax.axis_index("s")
    w = c * NS + s
    for k in range((STRIPE + 8) // 16):
        stage_v[pl.ds(k * 16, 16)] = jnp.zeros((16,), jnp.float32)
    for k in range(CH // 16):
        ones_v[pl.ds(k * 16, 16)] = jnp.full((16,), 1.0, jnp.float32)
    pltpu.sync_copy(stage_v.at[pl.ds(0, STRIPE)],
                    acc_sh.at[pl.ds(s * STRIPE, STRIPE)])
    pltpu.sync_copy(dst_hbm.at[pl.ds(w * TCH, TCH)], idx_v)
    plsc.subcore_barrier()

    def body(j, carry):
        pltpu.sync_copy(ones_v, acc_sh.at[idx_v.at[j]], add=True)
        return carry

    lax.fori_loop(0, TCH, body, 0)
    plsc.subcore_barrier()
    pltpu.sync_copy(acc_sh.at[pl.ds(s * STRIPE, STRIPE)],
                    stage_v.at[pl.ds(0, STRIPE)])
    pltpu.sync_copy(stage_v.at[pl.ds(0, STRIPE)],
                    out_hbm.at[pl.ds(c * N_PAD + s * STRIPE, STRIPE)])


NBUF = 4
OUTER = TCH // NBUF


@functools.partial(
    pl.kernel,
    out_type=jax.ShapeDtypeStruct((NC, N_PAD, F), jnp.float32),
    mesh=_mesh,
    scratch_types=[
        pltpu.VMEM((TCH, CH), jnp.int32),
        pltpu.VMEM((TCH, CH), jnp.int32),
        pltpu.VMEM((NBUF, CH, F), jnp.float32),
        pltpu.VMEM((STRIPE, F), jnp.float32),
        pltpu.VMEM_SHARED((N_PAD, F), jnp.float32),
        pltpu.VMEM_SHARED((N_PAD, F), jnp.float32),
        pltpu.SemaphoreType.DMA,
    ] + [pltpu.SemaphoreType.DMA] * (2 * NBUF),
    compiler_params=_sc_params,
)
def _scat_kernel(g_hbm, src_hbm, dst_hbm, out_hbm,
                 sidx, didx, rows, stage_v, acc_sh, g_sh, isem, *bsems):
    gsems = bsems[:NBUF]
    ssems = bsems[NBUF:]
    c = lax.axis_index("c")
    s = lax.axis_index("s")
    w = c * NS + s

    icopy_s = pltpu.async_copy(src_hbm.at[pl.ds(w * TCH, TCH)], sidx, isem)
    icopy_d = pltpu.async_copy(dst_hbm.at[pl.ds(w * TCH, TCH)], didx, isem)
    pltpu.sync_copy(g_hbm.at[pl.ds(s * STRIPE, STRIPE), :], stage_v)
    pltpu.sync_copy(stage_v, g_sh.at[pl.ds(s * STRIPE, STRIPE)])
    zv = jnp.zeros((F,), jnp.float32)

    def zbody(i, carry):
        for r in range(8):
            stage_v[i * 8 + r, :] = zv
        return carry

    lax.fori_loop(0, STRIPE // 8, zbody, 0)
    pltpu.sync_copy(stage_v, acc_sh.at[pl.ds(s * STRIPE, STRIPE)])
    icopy_s.wait()
    icopy_d.wait()
    plsc.subcore_barrier()

    for b in range(NBUF):
        pltpu.async_copy(g_sh.at[sidx.at[b]], rows.at[b], gsems[b])

    def body(jj, carry):
        for b in range(NBUF):
            j = jj * NBUF + b
            pltpu.make_async_copy(g_sh.at[sidx.at[j]], rows.at[b],
                                  gsems[b]).wait()
            pltpu.async_copy(rows.at[b], acc_sh.at[didx.at[j]], ssems[b],
                             add=True)
            pltpu.make_async_copy(rows.at[b], acc_sh.at[didx.at[j]],
                                  ssems[b]).wait()
            pltpu.async_copy(g_sh.at[sidx.at[j + NBUF]], rows.at[b],
                             gsems[b])
        return carry

    lax.fori_loop(0, OUTER - 1, body, 0)
    for b in range(NBUF):
        j = (OUTER - 1) * NBUF + b
        pltpu.make_async_copy(g_sh.at[sidx.at[j]], rows.at[b],
                              gsems[b]).wait()
        pltpu.async_copy(rows.at[b], acc_sh.at[didx.at[j]], ssems[b],
                         add=True)
    for b in range(NBUF):
        j = (OUTER - 1) * NBUF + b
        pltpu.make_async_copy(rows.at[b], acc_sh.at[didx.at[j]],
                              ssems[b]).wait()
    plsc.subcore_barrier()
    pltpu.sync_copy(acc_sh.at[pl.ds(s * STRIPE, STRIPE)], stage_v)
    pltpu.sync_copy(stage_v, out_hbm.at[c, pl.ds(s * STRIPE, STRIPE), :])


def _prep_body(deg_ref, x_ref, w1_ref, g_ref, dinv_ref):
    deg = deg_ref[0, :] + deg_ref[1, :] + 1.0
    dinv = lax.rsqrt(deg)
    dinv_ref[...] = dinv
    h = jnp.dot(x_ref[...], w1_ref[...], preferred_element_type=jnp.float32)
    g_ref[...] = h * dinv[:, None]


def _mid_body(s_ref, g_ref, dinv_ref, b_ref, w2_ref, out_ref):
    tot = (s_ref[0] + s_ref[1] + g_ref[...]) * dinv_ref[...][:, None]
    h = jnp.maximum(tot + b_ref[...][None, :], 0.0)
    out_ref[...] = jnp.dot(h, w2_ref[...],
                           preferred_element_type=jnp.float32) * dinv_ref[...][:, None]


def _final_body(s_ref, g_ref, dinv_ref, b_ref, out_ref):
    logits = (s_ref[0] + s_ref[1] + g_ref[...]) * dinv_ref[...][:, None]
    logits = logits + b_ref[...][None, :]
    m = jnp.max(logits, axis=1, keepdims=True)
    z = logits - m
    out_ref[...] = z - jnp.log(jnp.sum(jnp.exp(z), axis=1, keepdims=True))


_prep_call = pl.pallas_call(
    _prep_body,
    out_shape=(jax.ShapeDtypeStruct((N_PAD, F), jnp.float32),
               jax.ShapeDtypeStruct((N_PAD,), jnp.float32)),
)

_mid_call = pl.pallas_call(
    _mid_body,
    out_shape=jax.ShapeDtypeStruct((N_PAD, F), jnp.float32),
)

_final_call = pl.pallas_call(
    _final_body,
    out_shape=jax.ShapeDtypeStruct((N_PAD, F), jnp.float32),
)


def kernel(x, edge_index, W1, b1, W2, b2):
    src = edge_index[0]
    dst = edge_index[1]
    pad = jnp.full((E_PAD - E,), N, jnp.int32)
    src_p = jnp.concatenate([src, pad]).reshape(NCH_TOT, CH)
    dst_p = jnp.concatenate([dst, pad]).reshape(NCH_TOT, CH)
    x_p = jnp.pad(x, ((0, N_PAD - N), (0, 0)))

    deg2 = _deg_kernel(dst_p).reshape(NC, N_PAD)
    g1, dinv = _prep_call(deg2, x_p, W1)
    s1 = _scat_kernel(g1, src_p, dst_p)
    g2 = _mid_call(s1, g1, dinv, b1, W2)
    s2 = _scat_kernel(g2, src_p, dst_p)
    out = _final_call(s2, g2, dinv, b2)
    return out[:N]


# speedup vs baseline: 63.4592x; 1.0002x over previous
import functools

import jax
import jax.numpy as jnp
from jax import lax
from jax.experimental import pallas as pl
from jax.experimental.pallas import tpu as pltpu
from jax.experimental.pallas import tpu_sc as plsc

N = 10000
E = 320000
D = 128
F = 16

NC = 2
NS = 16
NW = NC * NS

CH = 128
TCH = 80
E_PAD = NW * TCH * CH
NCH_TOT = E_PAD // CH
N_PAD = 10112
STRIPE = N_PAD // NS
LASTN = N - (NS - 1) * STRIPE

_mesh = plsc.VectorSubcoreMesh(core_axis_name="c", subcore_axis_name="s")
_sc_params = pltpu.CompilerParams(use_tc_tiling_on_sc=False)


@functools.partial(
    pl.kernel,
    out_type=jax.ShapeDtypeStruct((NC * N_PAD,), jnp.float32),
    mesh=_mesh,
    scratch_types=[
        pltpu.VMEM((TCH, CH), jnp.int32),
        pltpu.VMEM((CH,), jnp.float32),
        pltpu.VMEM((STRIPE + 8,), jnp.float32),
        pltpu.VMEM_SHARED((N_PAD,), jnp.float32),
        pltpu.SemaphoreType.DMA,
    ],
    compiler_params=_sc_params,
)
def _deg_kernel(dst_hbm, out_hbm, idx_v, ones_v, stage_v, acc_sh, dsem):
    c = lax.axis_index("c")
    s = lax.axis_index("s")
    w = c * NS + s
    icopy = pltpu.async_copy(dst_hbm.at[pl.ds(w * TCH, TCH)], idx_v, dsem)
    for k in range((STRIPE + 8) // 16):
        stage_v[pl.ds(k * 16, 16)] = jnp.zeros((16,), jnp.float32)
    for k in range(CH // 16):
        ones_v[pl.ds(k * 16, 16)] = jnp.full((16,), 1.0, jnp.float32)
    pltpu.sync_copy(stage_v.at[pl.ds(0, STRIPE)],
                    acc_sh.at[pl.ds(s * STRIPE, STRIPE)])
    icopy.wait()
    plsc.subcore_barrier()

    DINF = 8
    for j in range(DINF):
        pltpu.async_copy(ones_v, acc_sh.at[idx_v.at[j]], dsem, add=True)

    def body(j, carry):
        pltpu.make_async_copy(ones_v, acc_sh.at[idx_v.at[j]], dsem).wait()
        pltpu.async_copy(ones_v, acc_sh.at[idx_v.at[j + DINF]], dsem,
                         add=True)
        return carry

    lax.fori_loop(0, TCH - DINF, body, 0)

    def drain(j, carry):
        pltpu.make_async_copy(ones_v, acc_sh.at[idx_v.at[j]], dsem).wait()
        return carry

    lax.fori_loop(TCH - DINF, TCH, drain, 0)
    plsc.subcore_barrier()
    pltpu.sync_copy(acc_sh.at[pl.ds(s * STRIPE, STRIPE)],
                    stage_v.at[pl.ds(0, STRIPE)])
    pltpu.sync_copy(stage_v.at[pl.ds(0, STRIPE)],
                    out_hbm.at[pl.ds(c * N_PAD + s * STRIPE, STRIPE)])


NBUF = 8
AHEAD = 4
STEADY = (TCH - 2 * AHEAD) // NBUF


@functools.partial(
    pl.kernel,
    out_type=jax.ShapeDtypeStruct((NC, N, F), jnp.float32),
    mesh=_mesh,
    scratch_types=[
        pltpu.VMEM((TCH, CH), jnp.int32),
        pltpu.VMEM((TCH, CH), jnp.int32),
        pltpu.VMEM((NBUF, CH, F), jnp.float32),
        pltpu.VMEM((STRIPE, F), jnp.float32),
        pltpu.VMEM_SHARED((N_PAD, F), jnp.float32),
        pltpu.VMEM_SHARED((N_PAD, F), jnp.float32),
        pltpu.SemaphoreType.DMA,
    ] + [pltpu.SemaphoreType.DMA] * (2 * NBUF),
    compiler_params=_sc_params,
)
def _scat_kernel(g_hbm, src_hbm, dst_hbm, out_hbm,
                 sidx, didx, rows, stage_v, acc_sh, g_sh, isem, *bsems):
    gsems = bsems[:NBUF]
    ssems = bsems[NBUF:]
    c = lax.axis_index("c")
    s = lax.axis_index("s")
    w = c * NS + s
    zv = jnp.zeros((F,), jnp.float32)

    icopy_s = pltpu.async_copy(src_hbm.at[pl.ds(w * TCH, TCH)], sidx, isem)
    icopy_d = pltpu.async_copy(dst_hbm.at[pl.ds(w * TCH, TCH)], didx, isem)

    @pl.when(s < NS - 1)
    def _():
        pltpu.sync_copy(g_hbm.at[pl.ds(s * STRIPE, STRIPE), :], stage_v)

    @pl.when(s == NS - 1)
    def _():
        pltpu.sync_copy(g_hbm.at[pl.ds((NS - 1) * STRIPE, LASTN), :],
                        stage_v.at[pl.ds(0, LASTN)])

        def ztail(i, carry):
            stage_v[LASTN + i, :] = zv
            return carry

        lax.fori_loop(0, STRIPE - LASTN, ztail, 0)

    pltpu.sync_copy(stage_v, g_sh.at[pl.ds(s * STRIPE, STRIPE)])

    def zbody(i, carry):
        for r in range(8):
            stage_v[i * 8 + r, :] = zv
        return carry

    lax.fori_loop(0, STRIPE // 8, zbody, 0)
    pltpu.sync_copy(stage_v, acc_sh.at[pl.ds(s * STRIPE, STRIPE)])
    icopy_s.wait()
    icopy_d.wait()
    plsc.subcore_barrier()

    def gather(j, b):
        pltpu.async_copy(g_sh.at[sidx.at[j]], rows.at[b], gsems[b])

    def wait_gather(j, b):
        pltpu.make_async_copy(g_sh.at[sidx.at[j]], rows.at[b],
                              gsems[b]).wait()

    def scatter(j, b):
        pltpu.async_copy(rows.at[b], acc_sh.at[didx.at[j]], ssems[b],
                         add=True)

    def wait_scatter(j, b):
        pltpu.make_async_copy(rows.at[b], acc_sh.at[didx.at[j]],
                              ssems[b]).wait()

    for j in range(AHEAD):
        gather(j, j % NBUF)
    for j in range(AHEAD):
        wait_gather(j, j % NBUF)
        scatter(j, j % NBUF)
        gather(j + AHEAD, (j + AHEAD) % NBUF)

    def body(jj, carry):
        for b in range(NBUF):
            j = AHEAD + jj * NBUF + b
            bj = (AHEAD + b) % NBUF
            wait_gather(j, bj)
            scatter(j, bj)
            wait_scatter(j - AHEAD, b)
            gather(j + AHEAD, b)
        return carry

    lax.fori_loop(0, STEADY, body, 0)
    for jo in range(AHEAD):
        j = TCH - AHEAD + jo
        bj = j % NBUF
        wait_gather(j, bj)
        scatter(j, bj)
        wait_scatter(j - AHEAD, (j - AHEAD) % NBUF)
    for jo in range(AHEAD):
        j = TCH - AHEAD + jo
        wait_scatter(j, j % NBUF)

    plsc.subcore_barrier()

    @pl.when(s < NS - 1)
    def _():
        pltpu.sync_copy(acc_sh.at[pl.ds(s * STRIPE, STRIPE)], stage_v)
        pltpu.sync_copy(stage_v, out_hbm.at[c, pl.ds(s * STRIPE, STRIPE), :])

    @pl.when(s == NS - 1)
    def _():
        pltpu.sync_copy(acc_sh.at[pl.ds((NS - 1) * STRIPE, LASTN)],
                        stage_v.at[pl.ds(0, LASTN)])
        pltpu.sync_copy(stage_v.at[pl.ds(0, LASTN)],
                        out_hbm.at[c, pl.ds((NS - 1) * STRIPE, LASTN), :])


def _prep_body(deg_ref, x_ref, w1_ref, g_ref, dinvr_ref):
    deg = deg_ref[pl.ds(0, N)] + deg_ref[pl.ds(N_PAD, N)] + 1.0
    dinvr = jnp.broadcast_to(lax.rsqrt(deg)[:, None], (N, F))
    dinvr_ref[...] = dinvr
    h = jnp.dot(x_ref[...], w1_ref[...], preferred_element_type=jnp.float32)
    g_ref[...] = h * dinvr


def _mid_body(s_ref, g_ref, dinvr_ref, b_ref, w2_ref, out_ref):
    tot = (s_ref[0] + s_ref[1] + g_ref[...]) * dinvr_ref[...]
    h = jnp.maximum(tot + b_ref[...][None, :], 0.0)
    out_ref[...] = jnp.dot(h, w2_ref[...],
                           preferred_element_type=jnp.float32) * dinvr_ref[...]


def _final_body(s_ref, g_ref, dinvr_ref, b_ref, out_ref):
    logits = (s_ref[0] + s_ref[1] + g_ref[...]) * dinvr_ref[...]
    logits = logits + b_ref[...][None, :]
    m = jnp.max(logits, axis=1, keepdims=True)
    z = logits - m
    out_ref[...] = z - jnp.log(jnp.sum(jnp.exp(z), axis=1, keepdims=True))


_prep_call = pl.pallas_call(
    _prep_body,
    out_shape=(jax.ShapeDtypeStruct((N, F), jnp.float32),
               jax.ShapeDtypeStruct((N, F), jnp.float32)),
)

_mid_call = pl.pallas_call(
    _mid_body,
    out_shape=jax.ShapeDtypeStruct((N, F), jnp.float32),
)

_final_call = pl.pallas_call(
    _final_body,
    out_shape=jax.ShapeDtypeStruct((N, F), jnp.float32),
)


def kernel(x, edge_index, W1, b1, W2, b2):
    src = edge_index[0]
    dst = edge_index[1]
    pad = jnp.full((E_PAD - E,), N, jnp.int32)
    src_p = jnp.concatenate([src, pad]).reshape(NCH_TOT, CH)
    dst_p = jnp.concatenate([dst, pad]).reshape(NCH_TOT, CH)

    deg2 = _deg_kernel(dst_p)
    g1, dinvr = _prep_call(deg2, x, W1)
    s1 = _scat_kernel(g1, src_p, dst_p)
    g2 = _mid_call(s1, g1, dinvr, b1, W2)
    s2 = _scat_kernel(g2, src_p, dst_p)
    return _final_call(s2, g2, dinvr, b2)


# speedup vs baseline: 65.9974x; 1.0400x over previous
import functools

import jax
import jax.numpy as jnp
from jax import lax
from jax.experimental import pallas as pl
from jax.experimental.pallas import tpu as pltpu
from jax.experimental.pallas import tpu_sc as plsc

N = 10000
E = 320000
D = 128
F = 16

NC = 2
NS = 16

CH = 128
TCH0 = 96
TCH1 = 64
NCH_TOT = NS * (TCH0 + TCH1)
E_PAD = NCH_TOT * CH
N_PAD = 10112
STRIPE = N_PAD // NS
LASTN = N - (NS - 1) * STRIPE

_mesh = plsc.VectorSubcoreMesh(core_axis_name="c", subcore_axis_name="s")
_sc_params = pltpu.CompilerParams(use_tc_tiling_on_sc=False)


def _chunk_base(c, s):
    return jnp.where(c == 0, s * TCH0, NS * TCH0 + s * TCH1)


@functools.partial(
    pl.kernel,
    out_type=jax.ShapeDtypeStruct((NC * N_PAD,), jnp.float32),
    mesh=_mesh,
    scratch_types=[
        pltpu.VMEM((TCH0, CH), jnp.int32),
        pltpu.VMEM((CH,), jnp.float32),
        pltpu.VMEM((STRIPE + 8,), jnp.float32),
        pltpu.VMEM_SHARED((N_PAD,), jnp.float32),
        pltpu.SemaphoreType.DMA,
        pltpu.SemaphoreType.DMA,
    ],
    compiler_params=_sc_params,
)
def _deg_kernel(dst_hbm, out_hbm, idx_v, ones_v, stage_v, acc_sh, isem, dsem):
    c = lax.axis_index("c")
    s = lax.axis_index("s")
    base = _chunk_base(c, s)
    tch = jnp.where(c == 0, TCH0, TCH1)

    @pl.when(c == 0)
    def _():
        pltpu.async_copy(dst_hbm.at[pl.ds(base, TCH0)], idx_v, isem)

    @pl.when(c != 0)
    def _():
        pltpu.async_copy(dst_hbm.at[pl.ds(base, TCH1)],
                         idx_v.at[pl.ds(0, TCH1)], isem)

    for k in range((STRIPE + 8) // 16):
        stage_v[pl.ds(k * 16, 16)] = jnp.zeros((16,), jnp.float32)
    for k in range(CH // 16):
        ones_v[pl.ds(k * 16, 16)] = jnp.full((16,), 1.0, jnp.float32)
    pltpu.sync_copy(stage_v.at[pl.ds(0, STRIPE)],
                    acc_sh.at[pl.ds(s * STRIPE, STRIPE)])

    def iwait(n):
        pltpu.make_async_copy(dst_hbm.at[pl.ds(base, n)],
                              idx_v.at[pl.ds(0, n)], isem).wait()

    @pl.when(c == 0)
    def _():
        iwait(TCH0)

    @pl.when(c != 0)
    def _():
        iwait(TCH1)

    plsc.subcore_barrier()

    DINF = 8
    for j in range(DINF):
        pltpu.async_copy(ones_v, acc_sh.at[idx_v.at[j]], dsem, add=True)

    def body(j, carry):
        pltpu.make_async_copy(ones_v, acc_sh.at[idx_v.at[j]], dsem).wait()
        pltpu.async_copy(ones_v, acc_sh.at[idx_v.at[j + DINF]], dsem,
                         add=True)
        return carry

    lax.fori_loop(0, tch - DINF, body, 0)

    def drain(j, carry):
        pltpu.make_async_copy(ones_v, acc_sh.at[idx_v.at[j]], dsem).wait()
        return carry

    lax.fori_loop(tch - DINF, tch, drain, 0)
    plsc.subcore_barrier()
    pltpu.sync_copy(acc_sh.at[pl.ds(s * STRIPE, STRIPE)],
                    stage_v.at[pl.ds(0, STRIPE)])
    pltpu.sync_copy(stage_v.at[pl.ds(0, STRIPE)],
                    out_hbm.at[pl.ds(c * N_PAD + s * STRIPE, STRIPE)])


NBUF = 8
AHEAD = 4
STEADY0 = (TCH0 - 2 * AHEAD) // NBUF
STEADY1 = (TCH1 - 2 * AHEAD) // NBUF


@functools.partial(
    pl.kernel,
    out_type=jax.ShapeDtypeStruct((NC, N, F), jnp.float32),
    mesh=_mesh,
    scratch_types=[
        pltpu.VMEM((TCH0, CH), jnp.int32),
        pltpu.VMEM((TCH0, CH), jnp.int32),
        pltpu.VMEM((NBUF, CH, F), jnp.float32),
        pltpu.VMEM((STRIPE, F), jnp.float32),
        pltpu.VMEM_SHARED((N_PAD, F), jnp.float32),
        pltpu.VMEM_SHARED((N_PAD, F), jnp.float32),
        pltpu.SemaphoreType.DMA,
    ] + [pltpu.SemaphoreType.DMA] * (2 * NBUF),
    compiler_params=_sc_params,
)
def _scat_kernel(g_hbm, src_hbm, dst_hbm, out_hbm,
                 sidx, didx, rows, stage_v, acc_sh, g_sh, isem, *bsems):
    gsems = bsems[:NBUF]
    ssems = bsems[NBUF:]
    c = lax.axis_index("c")
    s = lax.axis_index("s")
    base = _chunk_base(c, s)
    zv = jnp.zeros((F,), jnp.float32)

    @pl.when(c == 0)
    def _():
        pltpu.async_copy(src_hbm.at[pl.ds(base, TCH0)], sidx, isem)
        pltpu.async_copy(dst_hbm.at[pl.ds(base, TCH0)], didx, isem)

    @pl.when(c != 0)
    def _():
        pltpu.async_copy(src_hbm.at[pl.ds(base, TCH1)],
                         sidx.at[pl.ds(0, TCH1)], isem)
        pltpu.async_copy(dst_hbm.at[pl.ds(base, TCH1)],
                         didx.at[pl.ds(0, TCH1)], isem)

    @pl.when(s < NS - 1)
    def _():
        pltpu.sync_copy(g_hbm.at[pl.ds(s * STRIPE, STRIPE), :], stage_v)

    @pl.when(s == NS - 1)
    def _():
        pltpu.sync_copy(g_hbm.at[pl.ds((NS - 1) * STRIPE, LASTN), :],
                        stage_v.at[pl.ds(0, LASTN)])

        def ztail(i, carry):
            stage_v[LASTN + i, :] = zv
            return carry

        lax.fori_loop(0, STRIPE - LASTN, ztail, 0)

    pltpu.sync_copy(stage_v, g_sh.at[pl.ds(s * STRIPE, STRIPE)])

    def zbody(i, carry):
        for r in range(8):
            stage_v[i * 8 + r, :] = zv
        return carry

    lax.fori_loop(0, STRIPE // 8, zbody, 0)
    pltpu.sync_copy(stage_v, acc_sh.at[pl.ds(s * STRIPE, STRIPE)])

    def iwait(ref, vref, n):
        pltpu.make_async_copy(ref.at[pl.ds(base, n)],
                              vref.at[pl.ds(0, n)], isem).wait()

    @pl.when(c == 0)
    def _():
        iwait(src_hbm, sidx, TCH0)
        iwait(dst_hbm, didx, TCH0)

    @pl.when(c != 0)
    def _():
        iwait(src_hbm, sidx, TCH1)
        iwait(dst_hbm, didx, TCH1)

    plsc.subcore_barrier()

    def gather(j, b):
        pltpu.async_copy(g_sh.at[sidx.at[j]], rows.at[b], gsems[b])

    def wait_gather(j, b):
        pltpu.make_async_copy(g_sh.at[sidx.at[j]], rows.at[b],
                              gsems[b]).wait()

    def scatter(j, b):
        pltpu.async_copy(rows.at[b], acc_sh.at[didx.at[j]], ssems[b],
                         add=True)

    def wait_scatter(j, b):
        pltpu.make_async_copy(rows.at[b], acc_sh.at[didx.at[j]],
                              ssems[b]).wait()

    tch = jnp.where(c == 0, TCH0, TCH1)
    steady = jnp.where(c == 0, STEADY0, STEADY1)

    for j in range(AHEAD):
        gather(j, j % NBUF)
    for j in range(AHEAD):
        wait_gather(j, j % NBUF)
        scatter(j, j % NBUF)
        gather(j + AHEAD, (j + AHEAD) % NBUF)

    def body(jj, carry):
        for b in range(NBUF):
            j = AHEAD + jj * NBUF + b
            bj = (AHEAD + b) % NBUF
            wait_gather(j, bj)
            scatter(j, bj)
            wait_scatter(j - AHEAD, b)
            gather(j + AHEAD, b)
        return carry

    lax.fori_loop(0, steady, body, 0)
    for jo in range(AHEAD):
        j = tch - AHEAD + jo
        bj = (AHEAD + jo) % NBUF
        wait_gather(j, bj)
        scatter(j, bj)
        wait_scatter(j - AHEAD, jo % NBUF)
    for jo in range(AHEAD):
        j = tch - AHEAD + jo
        wait_scatter(j, (AHEAD + jo) % NBUF)

    plsc.subcore_barrier()

    @pl.when(s < NS - 1)
    def _():
        pltpu.sync_copy(acc_sh.at[pl.ds(s * STRIPE, STRIPE)], stage_v)
        pltpu.sync_copy(stage_v, out_hbm.at[c, pl.ds(s * STRIPE, STRIPE), :])

    @pl.when(s == NS - 1)
    def _():
        pltpu.sync_copy(acc_sh.at[pl.ds((NS - 1) * STRIPE, LASTN)],
                        stage_v.at[pl.ds(0, LASTN)])
        pltpu.sync_copy(stage_v.at[pl.ds(0, LASTN)],
                        out_hbm.at[c, pl.ds((NS - 1) * STRIPE, LASTN), :])


def _h1_body(x_ref, w1_ref, h_ref):
    h_ref[...] = jnp.dot(x_ref[...], w1_ref[...],
                         preferred_element_type=jnp.float32)


def _scale_body(deg_ref, h_ref, g_ref, dinvr_ref):
    deg = deg_ref[pl.ds(0, N)] + deg_ref[pl.ds(N_PAD, N)] + 1.0
    dinvr = jnp.broadcast_to(lax.rsqrt(deg)[:, None], (N, F))
    dinvr_ref[...] = dinvr
    g_ref[...] = h_ref[...] * dinvr


def _mid_body(s_ref, g_ref, dinvr_ref, b_ref, w2_ref, out_ref):
    tot = (s_ref[0] + s_ref[1] + g_ref[...]) * dinvr_ref[...]
    h = jnp.maximum(tot + b_ref[...][None, :], 0.0)
    out_ref[...] = jnp.dot(h, w2_ref[...],
                           preferred_element_type=jnp.float32) * dinvr_ref[...]


def _final_body(s_ref, g_ref, dinvr_ref, b_ref, out_ref):
    logits = (s_ref[0] + s_ref[1] + g_ref[...]) * dinvr_ref[...]
    logits = logits + b_ref[...][None, :]
    m = jnp.max(logits, axis=1, keepdims=True)
    z = logits - m
    out_ref[...] = z - jnp.log(jnp.sum(jnp.exp(z), axis=1, keepdims=True))


_h1_call = pl.pallas_call(
    _h1_body,
    out_shape=jax.ShapeDtypeStruct((N, F), jnp.float32),
)

_scale_call = pl.pallas_call(
    _scale_body,
    out_shape=(jax.ShapeDtypeStruct((N, F), jnp.float32),
               jax.ShapeDtypeStruct((N, F), jnp.float32)),
)

_mid_call = pl.pallas_call(
    _mid_body,
    out_shape=jax.ShapeDtypeStruct((N, F), jnp.float32),
)

_final_call = pl.pallas_call(
    _final_body,
    out_shape=jax.ShapeDtypeStruct((N, F), jnp.float32),
)


def kernel(x, edge_index, W1, b1, W2, b2):
    src = edge_index[0]
    dst = edge_index[1]
    pad = jnp.full((E_PAD - E,), N, jnp.int32)
    src_p = jnp.concatenate([src, pad]).reshape(NCH_TOT, CH)
    dst_p = jnp.concatenate([dst, pad]).reshape(NCH_TOT, CH)

    h1 = _h1_call(x, W1)
    deg2 = _deg_kernel(dst_p)
    g1, dinvr = _scale_call(deg2, h1)
    s1 = _scat_kernel(g1, src_p, dst_p)
    g2 = _mid_call(s1, g1, dinvr, b1, W2)
    s2 = _scat_kernel(g2, src_p, dst_p)
    return _final_call(s2, g2, dinvr, b2)


# speedup vs baseline: 69.4370x; 1.0521x over previous
import functools

import jax
import jax.numpy as jnp
from jax import lax
from jax.experimental import pallas as pl
from jax.experimental.pallas import tpu as pltpu
from jax.experimental.pallas import tpu_sc as plsc

N = 10000
E = 320000
D = 128
F = 16

NC = 2
NS = 16

CH = 128
NCH = E // CH
XCH = 4
TCH_A = 96
TCH_B = 88
TCH_C = 64
N_PAD = 10112
STRIPE = N_PAD // NS
LASTN = N - (NS - 1) * STRIPE

_mesh = plsc.VectorSubcoreMesh(core_axis_name="c", subcore_axis_name="s")
_sc_params = pltpu.CompilerParams(use_tc_tiling_on_sc=False)


def _tile_chunks(c, s):
    tch = jnp.where(c == 0, jnp.where(s < 8, TCH_A, TCH_B), TCH_C)
    base = jnp.where(
        c == 0,
        jnp.where(s < 8, s * TCH_A, 8 * TCH_A + (s - 8) * TCH_B),
        8 * (TCH_A + TCH_B) + s * TCH_C,
    )
    return base, tch


def _stage_idx(hbm, vref, base, c, s, sem):
    def start(n):
        pltpu.async_copy(hbm.at[pl.ds(base, n)], vref.at[pl.ds(0, n)], sem)

    def wait(n):
        pltpu.make_async_copy(hbm.at[pl.ds(base, n)],
                              vref.at[pl.ds(0, n)], sem).wait()

    return start, wait


def _per_size(c, s, fn):
    @pl.when(jnp.logical_and(c == 0, s < 8))
    def _():
        fn(TCH_A)

    @pl.when(jnp.logical_and(c == 0, s >= 8))
    def _():
        fn(TCH_B)

    @pl.when(c != 0)
    def _():
        fn(TCH_C)


@functools.partial(
    pl.kernel,
    out_type=jax.ShapeDtypeStruct((NC * N_PAD,), jnp.float32),
    mesh=_mesh,
    scratch_types=[
        pltpu.VMEM((TCH_A, CH), jnp.int32),
        pltpu.VMEM((1, CH), jnp.int32),
        pltpu.VMEM((CH,), jnp.float32),
        pltpu.VMEM((STRIPE + 8,), jnp.float32),
        pltpu.VMEM_SHARED((N_PAD,), jnp.float32),
        pltpu.SemaphoreType.DMA,
        pltpu.SemaphoreType.DMA,
    ],
    compiler_params=_sc_params,
)
def _deg_kernel(dst_hbm, out_hbm, idx_v, xd_v, ones_v, stage_v, acc_sh,
                isem, dsem):
    c = lax.axis_index("c")
    s = lax.axis_index("s")
    base, tch = _tile_chunks(c, s)
    istart, iwait = _stage_idx(dst_hbm, idx_v, base, c, s, isem)
    _per_size(c, s, istart)

    for k in range((STRIPE + 8) // 16):
        stage_v[pl.ds(k * 16, 16)] = jnp.zeros((16,), jnp.float32)
    for k in range(CH // 16):
        ones_v[pl.ds(k * 16, 16)] = jnp.full((16,), 1.0, jnp.float32)
    pltpu.sync_copy(stage_v.at[pl.ds(0, STRIPE)],
                    acc_sh.at[pl.ds(s * STRIPE, STRIPE)])
    _per_size(c, s, iwait)
    plsc.subcore_barrier()

    DINF = 8
    for j in range(DINF):
        pltpu.async_copy(ones_v, acc_sh.at[idx_v.at[j]], dsem, add=True)

    def body(j, carry):
        pltpu.make_async_copy(ones_v, acc_sh.at[idx_v.at[j]], dsem).wait()
        pltpu.async_copy(ones_v, acc_sh.at[idx_v.at[j + DINF]], dsem,
                         add=True)
        return carry

    lax.fori_loop(0, tch - DINF, body, 0)

    def drain(j, carry):
        pltpu.make_async_copy(ones_v, acc_sh.at[idx_v.at[j]], dsem).wait()
        return carry

    lax.fori_loop(tch - DINF, tch, drain, 0)

    @pl.when(jnp.logical_and(c == 0, s < XCH))
    def _():
        pltpu.sync_copy(dst_hbm.at[pl.ds(NCH - XCH + s, 1), :], xd_v)
        pltpu.async_copy(ones_v, acc_sh.at[xd_v.at[0]], dsem, add=True)
        pltpu.make_async_copy(ones_v, acc_sh.at[xd_v.at[0]], dsem).wait()

    plsc.subcore_barrier()
    pltpu.sync_copy(acc_sh.at[pl.ds(s * STRIPE, STRIPE)],
                    stage_v.at[pl.ds(0, STRIPE)])
    pltpu.sync_copy(stage_v.at[pl.ds(0, STRIPE)],
                    out_hbm.at[pl.ds(c * N_PAD + s * STRIPE, STRIPE)])


NBUF = 8
AHEAD = 4


@functools.partial(
    pl.kernel,
    out_type=jax.ShapeDtypeStruct((NC, N, F), jnp.float32),
    mesh=_mesh,
    scratch_types=[
        pltpu.VMEM((TCH_A, CH), jnp.int32),
        pltpu.VMEM((TCH_A, CH), jnp.int32),
        pltpu.VMEM((1, CH), jnp.int32),
        pltpu.VMEM((1, CH), jnp.int32),
        pltpu.VMEM((NBUF, CH, F), jnp.float32),
        pltpu.VMEM((STRIPE, F), jnp.float32),
        pltpu.VMEM_SHARED((N_PAD, F), jnp.float32),
        pltpu.VMEM_SHARED((N_PAD, F), jnp.float32),
        pltpu.SemaphoreType.DMA,
    ] + [pltpu.SemaphoreType.DMA] * (2 * NBUF),
    compiler_params=_sc_params,
)
def _scat_kernel(g_hbm, src_hbm, dst_hbm, out_hbm,
                 sidx, didx, xs_v, xd_v, rows, stage_v, acc_sh, g_sh,
                 isem, *bsems):
    gsems = bsems[:NBUF]
    ssems = bsems[NBUF:]
    c = lax.axis_index("c")
    s = lax.axis_index("s")
    base, tch = _tile_chunks(c, s)
    steady = (tch - 2 * AHEAD) // NBUF
    zv = jnp.zeros((F,), jnp.float32)

    sstart, swait = _stage_idx(src_hbm, sidx, base, c, s, isem)
    dstart, dwait = _stage_idx(dst_hbm, didx, base, c, s, isem)
    _per_size(c, s, sstart)
    _per_size(c, s, dstart)

    @pl.when(s < NS - 1)
    def _():
        pltpu.sync_copy(g_hbm.at[pl.ds(s * STRIPE, STRIPE), :], stage_v)

    @pl.when(s == NS - 1)
    def _():
        pltpu.sync_copy(g_hbm.at[pl.ds((NS - 1) * STRIPE, LASTN), :],
                        stage_v.at[pl.ds(0, LASTN)])

        def ztail(i, carry):
            stage_v[LASTN + i, :] = zv
            return carry

        lax.fori_loop(0, STRIPE - LASTN, ztail, 0)

    pltpu.sync_copy(stage_v, g_sh.at[pl.ds(s * STRIPE, STRIPE)])

    def zbody(i, carry):
        for r in range(8):
            stage_v[i * 8 + r, :] = zv
        return carry

    lax.fori_loop(0, STRIPE // 8, zbody, 0)
    pltpu.sync_copy(stage_v, acc_sh.at[pl.ds(s * STRIPE, STRIPE)])
    _per_size(c, s, swait)
    _per_size(c, s, dwait)
    plsc.subcore_barrier()

    def gather(j, b):
        pltpu.async_copy(g_sh.at[sidx.at[j]], rows.at[b], gsems[b])

    def wait_gather(j, b):
        pltpu.make_async_copy(g_sh.at[sidx.at[j]], rows.at[b],
                              gsems[b]).wait()

    def scatter(j, b):
        pltpu.async_copy(rows.at[b], acc_sh.at[didx.at[j]], ssems[b],
                         add=True)

    def wait_scatter(j, b):
        pltpu.make_async_copy(rows.at[b], acc_sh.at[didx.at[j]],
                              ssems[b]).wait()

    for j in range(AHEAD):
        gather(j, j % NBUF)
    for j in range(AHEAD):
        wait_gather(j, j % NBUF)
        scatter(j, j % NBUF)
        gather(j + AHEAD, (j + AHEAD) % NBUF)

    def body(jj, carry):
        for b in range(NBUF):
            j = AHEAD + jj * NBUF + b
            bj = (AHEAD + b) % NBUF
            wait_gather(j, bj)
            scatter(j, bj)
            wait_scatter(j - AHEAD, b)
            gather(j + AHEAD, b)
        return carry

    lax.fori_loop(0, steady, body, 0)
    for jo in range(AHEAD):
        j = tch - AHEAD + jo
        bj = (AHEAD + jo) % NBUF
        wait_gather(j, bj)
        scatter(j, bj)
        wait_scatter(j - AHEAD, jo % NBUF)
    for jo in range(AHEAD):
        j = tch - AHEAD + jo
        wait_scatter(j, (AHEAD + jo) % NBUF)

    @pl.when(jnp.logical_and(c == 0, s < XCH))
    def _():
        pltpu.sync_copy(src_hbm.at[pl.ds(NCH - XCH + s, 1), :], xs_v)
        pltpu.sync_copy(dst_hbm.at[pl.ds(NCH - XCH + s, 1), :], xd_v)
        pltpu.async_copy(g_sh.at[xs_v.at[0]], rows.at[0], gsems[0])
        pltpu.make_async_copy(g_sh.at[xs_v.at[0]], rows.at[0],
                              gsems[0]).wait()
        pltpu.async_copy(rows.at[0], acc_sh.at[xd_v.at[0]], ssems[0],
                         add=True)
        pltpu.make_async_copy(rows.at[0], acc_sh.at[xd_v.at[0]],
                              ssems[0]).wait()

    plsc.subcore_barrier()

    @pl.when(s < NS - 1)
    def _():
        pltpu.sync_copy(acc_sh.at[pl.ds(s * STRIPE, STRIPE)], stage_v)
        pltpu.sync_copy(stage_v, out_hbm.at[c, pl.ds(s * STRIPE, STRIPE), :])

    @pl.when(s == NS - 1)
    def _():
        pltpu.sync_copy(acc_sh.at[pl.ds((NS - 1) * STRIPE, LASTN)],
                        stage_v.at[pl.ds(0, LASTN)])
        pltpu.sync_copy(stage_v.at[pl.ds(0, LASTN)],
                        out_hbm.at[c, pl.ds((NS - 1) * STRIPE, LASTN), :])


def _h1_body(x_ref, w1_ref, h_ref):
    h_ref[...] = jnp.dot(x_ref[...], w1_ref[...],
                         preferred_element_type=jnp.float32)


def _scale_body(deg_ref, h_ref, g_ref, dinv_ref):
    deg = deg_ref[pl.ds(0, N)] + deg_ref[pl.ds(N_PAD, N)] + 1.0
    dinv = lax.rsqrt(deg)
    dinv_ref[...] = dinv
    g_ref[...] = h_ref[...] * dinv[:, None]


def _mid_body(s_ref, g_ref, dinv_ref, b_ref, w2_ref, out_ref):
    dinv = dinv_ref[...][:, None]
    tot = (s_ref[0] + s_ref[1] + g_ref[...]) * dinv
    h = jnp.maximum(tot + b_ref[...][None, :], 0.0)
    out_ref[...] = jnp.dot(h, w2_ref[...],
                           preferred_element_type=jnp.float32) * dinv


def _final_body(s_ref, g_ref, dinv_ref, b_ref, out_ref):
    logits = (s_ref[0] + s_ref[1] + g_ref[...]) * dinv_ref[...][:, None]
    logits = logits + b_ref[...][None, :]
    m = jnp.max(logits, axis=1, keepdims=True)
    z = logits - m
    out_ref[...] = z - jnp.log(jnp.sum(jnp.exp(z), axis=1, keepdims=True))


_h1_call = pl.pallas_call(
    _h1_body,
    out_shape=jax.ShapeDtypeStruct((N, F), jnp.float32),
)

_scale_call = pl.pallas_call(
    _scale_body,
    out_shape=(jax.ShapeDtypeStruct((N, F), jnp.float32),
               jax.ShapeDtypeStruct((N,), jnp.float32)),
)

_mid_call = pl.pallas_call(
    _mid_body,
    out_shape=jax.ShapeDtypeStruct((N, F), jnp.float32),
)

_final_call = pl.pallas_call(
    _final_body,
    out_shape=jax.ShapeDtypeStruct((N, F), jnp.float32),
)


def kernel(x, edge_index, W1, b1, W2, b2):
    src_p = edge_index[0].reshape(NCH, CH)
    dst_p = edge_index[1].reshape(NCH, CH)

    h1 = _h1_call(x, W1)
    deg2 = _deg_kernel(dst_p)
    g1, dinv = _scale_call(deg2, h1)
    s1 = _scat_kernel(g1, src_p, dst_p)
    g2 = _mid_call(s1, g1, dinv, b1, W2)
    s2 = _scat_kernel(g2, src_p, dst_p)
    return _final_call(s2, g2, dinv, b2)


# speedup vs baseline: 72.2179x; 1.0400x over previous
import functools

import jax
import jax.numpy as jnp
from jax import lax
from jax.experimental import pallas as pl
from jax.experimental.pallas import tpu as pltpu
from jax.experimental.pallas import tpu_sc as plsc

N = 10000
E = 320000
D = 128
F = 16

NC = 2
NS = 16

CH = 128
NCH = E // CH
XCH = 4
TCH_A = 80
TCH_B = 72
PER_CORE = 12 * TCH_A + 4 * TCH_B
N_PAD = 10112
STRIPE = N_PAD // NS
LASTN = N - (NS - 1) * STRIPE

_mesh = plsc.VectorSubcoreMesh(core_axis_name="c", subcore_axis_name="s")
_sc_params = pltpu.CompilerParams(use_tc_tiling_on_sc=False)


def _tile_chunks(c, s):
    tch = jnp.where(s < 12, TCH_A, TCH_B)
    base = c * PER_CORE + jnp.where(s < 12, s * TCH_A,
                                    12 * TCH_A + (s - 12) * TCH_B)
    return base, tch


def _stage_idx(hbm, vref, base, c, s, sem):
    def start(n):
        pltpu.async_copy(hbm.at[pl.ds(base, n)], vref.at[pl.ds(0, n)], sem)

    def wait(n):
        pltpu.make_async_copy(hbm.at[pl.ds(base, n)],
                              vref.at[pl.ds(0, n)], sem).wait()

    return start, wait


def _per_size(c, s, fn):
    @pl.when(s < 12)
    def _():
        fn(TCH_A)

    @pl.when(s >= 12)
    def _():
        fn(TCH_B)


@functools.partial(
    pl.kernel,
    out_type=jax.ShapeDtypeStruct((NC * N_PAD,), jnp.float32),
    mesh=_mesh,
    scratch_types=[
        pltpu.VMEM((TCH_A, CH), jnp.int32),
        pltpu.VMEM((1, CH), jnp.int32),
        pltpu.VMEM((CH,), jnp.float32),
        pltpu.VMEM((STRIPE + 8,), jnp.float32),
        pltpu.VMEM_SHARED((N_PAD,), jnp.float32),
        pltpu.SemaphoreType.DMA,
        pltpu.SemaphoreType.DMA,
    ],
    compiler_params=_sc_params,
)
def _deg_kernel(dst_hbm, out_hbm, idx_v, xd_v, ones_v, stage_v, acc_sh,
                isem, dsem):
    c = lax.axis_index("c")
    s = lax.axis_index("s")
    base, tch = _tile_chunks(c, s)
    istart, iwait = _stage_idx(dst_hbm, idx_v, base, c, s, isem)
    _per_size(c, s, istart)

    for k in range((STRIPE + 8) // 16):
        stage_v[pl.ds(k * 16, 16)] = jnp.zeros((16,), jnp.float32)
    for k in range(CH // 16):
        ones_v[pl.ds(k * 16, 16)] = jnp.full((16,), 1.0, jnp.float32)
    pltpu.sync_copy(stage_v.at[pl.ds(0, STRIPE)],
                    acc_sh.at[pl.ds(s * STRIPE, STRIPE)])
    _per_size(c, s, iwait)
    plsc.subcore_barrier()

    DINF = 8
    for j in range(DINF):
        pltpu.async_copy(ones_v, acc_sh.at[idx_v.at[j]], dsem, add=True)

    def body(j, carry):
        pltpu.make_async_copy(ones_v, acc_sh.at[idx_v.at[j]], dsem).wait()
        pltpu.async_copy(ones_v, acc_sh.at[idx_v.at[j + DINF]], dsem,
                         add=True)
        return carry

    lax.fori_loop(0, tch - DINF, body, 0)

    def drain(j, carry):
        pltpu.make_async_copy(ones_v, acc_sh.at[idx_v.at[j]], dsem).wait()
        return carry

    lax.fori_loop(tch - DINF, tch, drain, 0)

    @pl.when(jnp.logical_and(c == 0, s < XCH))
    def _():
        pltpu.sync_copy(dst_hbm.at[pl.ds(NCH - XCH + s, 1), :], xd_v)
        pltpu.async_copy(ones_v, acc_sh.at[xd_v.at[0]], dsem, add=True)
        pltpu.make_async_copy(ones_v, acc_sh.at[xd_v.at[0]], dsem).wait()

    plsc.subcore_barrier()
    pltpu.sync_copy(acc_sh.at[pl.ds(s * STRIPE, STRIPE)],
                    stage_v.at[pl.ds(0, STRIPE)])
    pltpu.sync_copy(stage_v.at[pl.ds(0, STRIPE)],
                    out_hbm.at[pl.ds(c * N_PAD + s * STRIPE, STRIPE)])


NBUF = 8
AHEAD = 4


@functools.partial(
    pl.kernel,
    out_type=jax.ShapeDtypeStruct((NC, N, F), jnp.float32),
    mesh=_mesh,
    scratch_types=[
        pltpu.VMEM((TCH_A, CH), jnp.int32),
        pltpu.VMEM((TCH_A, CH), jnp.int32),
        pltpu.VMEM((1, CH), jnp.int32),
        pltpu.VMEM((1, CH), jnp.int32),
        pltpu.VMEM((NBUF, CH, F), jnp.float32),
        pltpu.VMEM((STRIPE, F), jnp.float32),
        pltpu.VMEM_SHARED((N_PAD, F), jnp.float32),
        pltpu.VMEM_SHARED((N_PAD, F), jnp.float32),
        pltpu.SemaphoreType.DMA,
    ] + [pltpu.SemaphoreType.DMA] * (2 * NBUF),
    compiler_params=_sc_params,
)
def _scat_kernel(g_hbm, src_hbm, dst_hbm, out_hbm,
                 sidx, didx, xs_v, xd_v, rows, stage_v, acc_sh, g_sh,
                 isem, *bsems):
    gsems = bsems[:NBUF]
    ssems = bsems[NBUF:]
    c = lax.axis_index("c")
    s = lax.axis_index("s")
    base, tch = _tile_chunks(c, s)
    steady = (tch - 2 * AHEAD) // NBUF
    zv = jnp.zeros((F,), jnp.float32)

    sstart, swait = _stage_idx(src_hbm, sidx, base, c, s, isem)
    dstart, dwait = _stage_idx(dst_hbm, didx, base, c, s, isem)
    _per_size(c, s, sstart)
    _per_size(c, s, dstart)

    @pl.when(s < NS - 1)
    def _():
        pltpu.sync_copy(g_hbm.at[pl.ds(s * STRIPE, STRIPE), :], stage_v)

    @pl.when(s == NS - 1)
    def _():
        pltpu.sync_copy(g_hbm.at[pl.ds((NS - 1) * STRIPE, LASTN), :],
                        stage_v.at[pl.ds(0, LASTN)])

        def ztail(i, carry):
            stage_v[LASTN + i, :] = zv
            return carry

        lax.fori_loop(0, STRIPE - LASTN, ztail, 0)

    pltpu.sync_copy(stage_v, g_sh.at[pl.ds(s * STRIPE, STRIPE)])

    def zbody(i, carry):
        for r in range(8):
            stage_v[i * 8 + r, :] = zv
        return carry

    lax.fori_loop(0, STRIPE // 8, zbody, 0)
    pltpu.sync_copy(stage_v, acc_sh.at[pl.ds(s * STRIPE, STRIPE)])
    _per_size(c, s, swait)
    _per_size(c, s, dwait)
    plsc.subcore_barrier()

    def gather(j, b):
        pltpu.async_copy(g_sh.at[sidx.at[j]], rows.at[b], gsems[b])

    def wait_gather(j, b):
        pltpu.make_async_copy(g_sh.at[sidx.at[j]], rows.at[b],
                              gsems[b]).wait()

    def scatter(j, b):
        pltpu.async_copy(rows.at[b], acc_sh.at[didx.at[j]], ssems[b],
                         add=True)

    def wait_scatter(j, b):
        pltpu.make_async_copy(rows.at[b], acc_sh.at[didx.at[j]],
                              ssems[b]).wait()

    for j in range(AHEAD):
        gather(j, j % NBUF)
    for j in range(AHEAD):
        wait_gather(j, j % NBUF)
        scatter(j, j % NBUF)
        gather(j + AHEAD, (j + AHEAD) % NBUF)

    def body(jj, carry):
        for b in range(NBUF):
            j = AHEAD + jj * NBUF + b
            bj = (AHEAD + b) % NBUF
            wait_gather(j, bj)
            scatter(j, bj)
            wait_scatter(j - AHEAD, b)
            gather(j + AHEAD, b)
        return carry

    lax.fori_loop(0, steady, body, 0)
    for jo in range(AHEAD):
        j = tch - AHEAD + jo
        bj = (AHEAD + jo) % NBUF
        wait_gather(j, bj)
        scatter(j, bj)
        wait_scatter(j - AHEAD, jo % NBUF)
    for jo in range(AHEAD):
        j = tch - AHEAD + jo
        wait_scatter(j, (AHEAD + jo) % NBUF)

    @pl.when(jnp.logical_and(c == 0, s < XCH))
    def _():
        pltpu.sync_copy(src_hbm.at[pl.ds(NCH - XCH + s, 1), :], xs_v)
        pltpu.sync_copy(dst_hbm.at[pl.ds(NCH - XCH + s, 1), :], xd_v)
        pltpu.async_copy(g_sh.at[xs_v.at[0]], rows.at[0], gsems[0])
        pltpu.make_async_copy(g_sh.at[xs_v.at[0]], rows.at[0],
                              gsems[0]).wait()
        pltpu.async_copy(rows.at[0], acc_sh.at[xd_v.at[0]], ssems[0],
                         add=True)
        pltpu.make_async_copy(rows.at[0], acc_sh.at[xd_v.at[0]],
                              ssems[0]).wait()

    plsc.subcore_barrier()

    @pl.when(s < NS - 1)
    def _():
        pltpu.sync_copy(acc_sh.at[pl.ds(s * STRIPE, STRIPE)], stage_v)
        pltpu.sync_copy(stage_v, out_hbm.at[c, pl.ds(s * STRIPE, STRIPE), :])

    @pl.when(s == NS - 1)
    def _():
        pltpu.sync_copy(acc_sh.at[pl.ds((NS - 1) * STRIPE, LASTN)],
                        stage_v.at[pl.ds(0, LASTN)])
        pltpu.sync_copy(stage_v.at[pl.ds(0, LASTN)],
                        out_hbm.at[c, pl.ds((NS - 1) * STRIPE, LASTN), :])


def _h1_body(x_ref, w1_ref, h_ref):
    h_ref[...] = jnp.dot(x_ref[...], w1_ref[...],
                         preferred_element_type=jnp.float32)


def _scale_body(deg_ref, h_ref, g_ref, dinv_ref):
    deg = deg_ref[pl.ds(0, N)] + deg_ref[pl.ds(N_PAD, N)] + 1.0
    dinv = lax.rsqrt(deg)
    dinv_ref[...] = dinv
    g_ref[...] = h_ref[...] * dinv[:, None]


def _mid_body(s_ref, g_ref, dinv_ref, b_ref, w2_ref, out_ref):
    dinv = dinv_ref[...][:, None]
    tot = (s_ref[0] + s_ref[1] + g_ref[...]) * dinv
    h = jnp.maximum(tot + b_ref[...][None, :], 0.0)
    out_ref[...] = jnp.dot(h, w2_ref[...],
                           preferred_element_type=jnp.float32) * dinv


def _final_body(s_ref, g_ref, dinv_ref, b_ref, out_ref):
    logits = (s_ref[0] + s_ref[1] + g_ref[...]) * dinv_ref[...][:, None]
    logits = logits + b_ref[...][None, :]
    m = jnp.max(logits, axis=1, keepdims=True)
    z = logits - m
    out_ref[...] = z - jnp.log(jnp.sum(jnp.exp(z), axis=1, keepdims=True))


_h1_call = pl.pallas_call(
    _h1_body,
    out_shape=jax.ShapeDtypeStruct((N, F), jnp.float32),
)

_scale_call = pl.pallas_call(
    _scale_body,
    out_shape=(jax.ShapeDtypeStruct((N, F), jnp.float32),
               jax.ShapeDtypeStruct((N,), jnp.float32)),
)

_mid_call = pl.pallas_call(
    _mid_body,
    out_shape=jax.ShapeDtypeStruct((N, F), jnp.float32),
)

_final_call = pl.pallas_call(
    _final_body,
    out_shape=jax.ShapeDtypeStruct((N, F), jnp.float32),
)


def kernel(x, edge_index, W1, b1, W2, b2):
    src_p = edge_index[0].reshape(NCH, CH)
    dst_p = edge_index[1].reshape(NCH, CH)

    h1 = _h1_call(x, W1)
    deg2 = _deg_kernel(dst_p)
    g1, dinv = _scale_call(deg2, h1)
    s1 = _scat_kernel(g1, src_p, dst_p)
    g2 = _mid_call(s1, g1, dinv, b1, W2)
    s2 = _scat_kernel(g2, src_p, dst_p)
    return _final_call(s2, g2, dinv, b2)
